# baseline (speedup 1.0000x reference)
import functools

import jax
import jax.numpy as jnp
from jax import lax
from jax.experimental import pallas as pl
from jax.experimental.pallas import tpu as pltpu
from jax.experimental.pallas import tpu_sc as plsc

N_NODES = 10000
N_EDGES = 320000
NUM_RELS = 4
NUM_BASES = 2

BLK = 128
N_TILES = 16
E_PAD = 327680
PAD_DST = 10008
N_ACC = 10112
ROWS_PER_TILE = N_ACC // N_TILES
MASK14 = (1 << 14) - 1


def _transform_body(relu_in, x_ref, bases_ref, wcomp_ref, out_ref):
    bf = jnp.bfloat16
    f32 = jnp.float32
    h = x_ref[...]
    if relu_in:
        h = jnp.maximum(h, 0.0)
    wc = wcomp_ref[0, 0].astype(bf).astype(f32)
    w = (wc[0] * bases_ref[0].astype(bf).astype(f32)
         + wc[1] * bases_ref[1].astype(bf).astype(f32))
    out_ref[0] = jnp.dot(h.astype(bf), w.astype(bf),
                         preferred_element_type=f32)


def _transform(h, bases, wcomp, relu_in):
    n, k = h.shape
    d_out = bases.shape[-1]
    nblk = 1000
    grid = (NUM_RELS, n // nblk)
    return pl.pallas_call(
        functools.partial(_transform_body, relu_in),
        grid=grid,
        in_specs=[
            pl.BlockSpec((nblk, k), lambda r, nb: (nb, 0)),
            pl.BlockSpec((NUM_BASES, k, d_out), lambda r, nb: (0, 0, 0)),
            pl.BlockSpec((1, 1, NUM_BASES), lambda r, nb: (r, 0, 0)),
        ],
        out_specs=pl.BlockSpec((1, nblk, d_out), lambda r, nb: (r, nb, 0)),
        out_shape=jax.ShapeDtypeStruct((NUM_RELS, n, d_out), jnp.float32),
    )(h, bases, wcomp.reshape(NUM_RELS, 1, NUM_BASES))


def _relu_body(x_ref, o_ref):
    o_ref[...] = jnp.maximum(x_ref[...], 0.0)


def _relu(x):
    return pl.pallas_call(
        _relu_body,
        out_shape=jax.ShapeDtypeStruct(x.shape, x.dtype),
    )(x)


NBUF = 4
AHEAD = 2


def _agg_body(dh, nsplit, q0, packed2, table, out, idxbuf, dstbuf, rows,
              acc, gsem, ssem):
    c = lax.axis_index("c")
    s = lax.axis_index("s")
    nblocks = idxbuf.shape[0]
    nchunk = dh // 16

    def zero_rows(i, _):
        rows[0, i // nchunk, pl.ds((i % nchunk) * 16, 16)] = jnp.zeros(
            (16,), jnp.float32)
        return 0

    lax.fori_loop(0, BLK * nchunk, zero_rows, 0)

    base = s * ROWS_PER_TILE
    nrep = ROWS_PER_TILE // BLK
    rem = ROWS_PER_TILE % BLK
    for rep in range(nrep):
        pltpu.sync_copy(rows.at[0], acc.at[pl.ds(base + rep * BLK, BLK)])
    if rem:
        pltpu.sync_copy(rows.at[0, pl.ds(0, rem)],
                        acc.at[pl.ds(base + nrep * BLK, rem)])
    plsc.subcore_barrier()

    roff = s * nblocks
    pltpu.sync_copy(packed2.at[pl.ds(roff, nblocks)], idxbuf)

    nvec = BLK // 16

    def calc_idx(i, _):
        j = i // nvec
        sl = pl.ds((i % nvec) * 16, 16)
        p = idxbuf[j, sl]
        dstbuf[j, sl] = p & MASK14
        idxbuf[j, sl] = ((p >> 28) * N_NODES
                         + ((p >> 14) & MASK14)) * nsplit + (q0 + c)
        return 0

    lax.fori_loop(0, nblocks * nvec, calc_idx, 0)

    def gather(j, b):
        pltpu.async_copy(table.at[idxbuf.at[j]], rows.at[b], gsem.at[b])

    def drain_gather(b):
        pltpu.make_async_copy(
            table.at[idxbuf.at[0]], rows.at[b], gsem.at[b]).wait()

    def scatter(j, b):
        pltpu.async_copy(rows.at[b], acc.at[dstbuf.at[j]], ssem.at[b],
                         add=True)

    def drain_scatter(b):
        pltpu.make_async_copy(
            rows.at[b], acc.at[dstbuf.at[0]], ssem.at[b]).wait()

    for b in range(AHEAD):
        gather(b, b)

    def outer(o, _):
        for k in range(NBUF):
            j = o * NBUF + k
            drain_gather(k)
            scatter(j, k)
            jn = j + AHEAD
            nxt = (k + AHEAD) % NBUF

            @pl.when(jn >= NBUF)
            def _():
                drain_scatter(nxt)

            @pl.when(jn < nblocks)
            def _():
                gather(jn, nxt)
        return 0

    lax.fori_loop(0, nblocks // NBUF, outer, 0)
    for i in range(AHEAD):
        drain_scatter((nblocks - AHEAD + i) % NBUF)
    plsc.subcore_barrier()

    pltpu.sync_copy(acc.at[pl.ds(base, ROWS_PER_TILE)],
                    out.at[c, pl.ds(base, ROWS_PER_TILE)])


def _aggregate(packed2, table, dh, nsplit, q0):
    nblocks = (E_PAD // BLK) // N_TILES
    mesh = plsc.VectorSubcoreMesh(core_axis_name="c", subcore_axis_name="s")
    return pl.kernel(
        functools.partial(_agg_body, dh, nsplit, q0),
        out_type=jax.ShapeDtypeStruct((2, N_ACC, dh), jnp.float32),
        mesh=mesh,
        scratch_types=[
            pltpu.VMEM((nblocks, BLK), jnp.int32),
            pltpu.VMEM((nblocks, BLK), jnp.int32),
            pltpu.VMEM((NBUF, BLK, dh), jnp.float32),
            pltpu.VMEM_SHARED((N_ACC, dh), jnp.float32),
            pltpu.SemaphoreType.DMA((NBUF,)),
            pltpu.SemaphoreType.DMA((NBUF,)),
        ],
        compiler_params=pltpu.CompilerParams(use_tc_tiling_on_sc=False),
    )(packed2, table)


def _layer(h, bases, wcomp, relu_in, packed2):
    d_out = bases.shape[-1]
    t = _transform(h, bases, wcomp, relu_in)
    table = t.reshape(NUM_RELS * N_NODES * 4, d_out // 4)
    agg_a = _aggregate(packed2, table, d_out // 4, 4, 0)
    agg_b = _aggregate(packed2, table, d_out // 4, 4, 2)
    return jnp.concatenate(
        [agg_a[0, :N_NODES], agg_a[1, :N_NODES],
         agg_b[0, :N_NODES], agg_b[1, :N_NODES]], axis=1)


def kernel(x, edge_index, edge_type, weight_in, w_comp_in, weight_h0,
           w_comp_h0, weight_out, w_comp_out):
    src = edge_index[0]
    dst = edge_index[1]
    packed = (
        jnp.left_shift(edge_type, 28)
        | jnp.left_shift(src, 14)
        | dst
    )
    pad = E_PAD - N_EDGES
    packed2 = jnp.concatenate(
        [packed, jnp.full((pad,), PAD_DST, jnp.int32)]).reshape(-1, BLK)

    h1 = _layer(x, weight_in, w_comp_in, False, packed2)
    h2 = _layer(h1, weight_h0, w_comp_h0, True, packed2)

    w3 = jnp.broadcast_to(weight_out, (NUM_BASES, weight_out.shape[1], 32))
    t3 = _transform(h2, w3, w_comp_out, True)
    table3 = t3.reshape(NUM_RELS * N_NODES * 2, 16)
    agg3 = _aggregate(packed2, table3, 16, 2, 0)
    out = _relu(agg3[0])
    return out[:N_NODES, 0:1]


# speedup vs baseline: 1.0145x; 1.0145x over previous
import functools

import jax
import jax.numpy as jnp
from jax import lax
from jax.experimental import pallas as pl
from jax.experimental.pallas import tpu as pltpu
from jax.experimental.pallas import tpu_sc as plsc

N_NODES = 10000
N_EDGES = 320000
NUM_RELS = 4
NUM_BASES = 2

BLK = 64
N_TILES = 16
E_PAD = 327680
PAD_DST = 10008
N_ACC = 10112
ROWS_PER_TILE = N_ACC // N_TILES
MASK14 = (1 << 14) - 1


def _transform_body(relu_in, x_ref, bases_ref, wcomp_ref, out_ref):
    bf = jnp.bfloat16
    f32 = jnp.float32
    h = x_ref[...]
    if relu_in:
        h = jnp.maximum(h, 0.0)
    wc = wcomp_ref[0, 0].astype(bf).astype(f32)
    w = (wc[0] * bases_ref[0].astype(bf).astype(f32)
         + wc[1] * bases_ref[1].astype(bf).astype(f32))
    out_ref[0] = jnp.dot(h.astype(bf), w.astype(bf),
                         preferred_element_type=f32)


def _transform(h, bases, wcomp, relu_in):
    n, k = h.shape
    d_out = bases.shape[-1]
    nblk = 1000
    grid = (NUM_RELS, n // nblk)
    return pl.pallas_call(
        functools.partial(_transform_body, relu_in),
        grid=grid,
        in_specs=[
            pl.BlockSpec((nblk, k), lambda r, nb: (nb, 0)),
            pl.BlockSpec((NUM_BASES, k, d_out), lambda r, nb: (0, 0, 0)),
            pl.BlockSpec((1, 1, NUM_BASES), lambda r, nb: (r, 0, 0)),
        ],
        out_specs=pl.BlockSpec((1, nblk, d_out), lambda r, nb: (r, nb, 0)),
        out_shape=jax.ShapeDtypeStruct((NUM_RELS, n, d_out), jnp.float32),
    )(h, bases, wcomp.reshape(NUM_RELS, 1, NUM_BASES))


def _relu_body(x_ref, o_ref):
    o_ref[...] = jnp.maximum(x_ref[...], 0.0)


def _relu(x):
    return pl.pallas_call(
        _relu_body,
        out_shape=jax.ShapeDtypeStruct(x.shape, x.dtype),
    )(x)


NBUF = 8
AHEAD = 4


def _agg_body(dh, nsplit, q0, packed2, table, out, idxbuf, dstbuf, rows,
              acc, gsem, ssem):
    c = lax.axis_index("c")
    s = lax.axis_index("s")
    nblocks = idxbuf.shape[0]
    nchunk = dh // 16

    def zero_rows(i, _):
        rows[0, i // nchunk, pl.ds((i % nchunk) * 16, 16)] = jnp.zeros(
            (16,), jnp.float32)
        return 0

    lax.fori_loop(0, BLK * nchunk, zero_rows, 0)

    base = s * ROWS_PER_TILE
    nrep = ROWS_PER_TILE // BLK
    rem = ROWS_PER_TILE % BLK
    for rep in range(nrep):
        pltpu.sync_copy(rows.at[0], acc.at[pl.ds(base + rep * BLK, BLK)])
    if rem:
        pltpu.sync_copy(rows.at[0, pl.ds(0, rem)],
                        acc.at[pl.ds(base + nrep * BLK, rem)])
    plsc.subcore_barrier()

    roff = s * nblocks
    pltpu.sync_copy(packed2.at[pl.ds(roff, nblocks)], idxbuf)

    nvec = BLK // 16

    def calc_idx(i, _):
        j = i // nvec
        sl = pl.ds((i % nvec) * 16, 16)
        p = idxbuf[j, sl]
        dstbuf[j, sl] = p & MASK14
        idxbuf[j, sl] = ((p >> 28) * N_NODES
                         + ((p >> 14) & MASK14)) * nsplit + (q0 + c)
        return 0

    lax.fori_loop(0, nblocks * nvec, calc_idx, 0)

    def gather(j, b):
        pltpu.async_copy(table.at[idxbuf.at[j]], rows.at[b], gsem.at[b])

    def drain_gather(b):
        pltpu.make_async_copy(
            table.at[idxbuf.at[0]], rows.at[b], gsem.at[b]).wait()

    def scatter(j, b):
        pltpu.async_copy(rows.at[b], acc.at[dstbuf.at[j]], ssem.at[b],
                         add=True)

    def drain_scatter(b):
        pltpu.make_async_copy(
            rows.at[b], acc.at[dstbuf.at[0]], ssem.at[b]).wait()

    for b in range(AHEAD):
        gather(b, b)

    def outer(o, _):
        for k in range(NBUF):
            j = o * NBUF + k
            drain_gather(k)
            scatter(j, k)
            jn = j + AHEAD
            nxt = (k + AHEAD) % NBUF

            @pl.when(jn >= NBUF)
            def _():
                drain_scatter(nxt)

            @pl.when(jn < nblocks)
            def _():
                gather(jn, nxt)
        return 0

    lax.fori_loop(0, nblocks // NBUF, outer, 0)
    for i in range(AHEAD):
        drain_scatter((nblocks - AHEAD + i) % NBUF)
    plsc.subcore_barrier()

    pltpu.sync_copy(acc.at[pl.ds(base, ROWS_PER_TILE)],
                    out.at[c, pl.ds(base, ROWS_PER_TILE)])


def _aggregate(packed2, table, dh, nsplit, q0):
    nblocks = (E_PAD // BLK) // N_TILES
    mesh = plsc.VectorSubcoreMesh(core_axis_name="c", subcore_axis_name="s")
    return pl.kernel(
        functools.partial(_agg_body, dh, nsplit, q0),
        out_type=jax.ShapeDtypeStruct((2, N_ACC, dh), jnp.float32),
        mesh=mesh,
        scratch_types=[
            pltpu.VMEM((nblocks, BLK), jnp.int32),
            pltpu.VMEM((nblocks, BLK), jnp.int32),
            pltpu.VMEM((NBUF, BLK, dh), jnp.float32),
            pltpu.VMEM_SHARED((N_ACC, dh), jnp.float32),
            pltpu.SemaphoreType.DMA((NBUF,)),
            pltpu.SemaphoreType.DMA((NBUF,)),
        ],
        compiler_params=pltpu.CompilerParams(use_tc_tiling_on_sc=False),
    )(packed2, table)


def _layer(h, bases, wcomp, relu_in, packed2):
    d_out = bases.shape[-1]
    t = _transform(h, bases, wcomp, relu_in)
    table = t.reshape(NUM_RELS * N_NODES * 4, d_out // 4)
    agg_a = _aggregate(packed2, table, d_out // 4, 4, 0)
    agg_b = _aggregate(packed2, table, d_out // 4, 4, 2)
    return jnp.concatenate(
        [agg_a[0, :N_NODES], agg_a[1, :N_NODES],
         agg_b[0, :N_NODES], agg_b[1, :N_NODES]], axis=1)


def kernel(x, edge_index, edge_type, weight_in, w_comp_in, weight_h0,
           w_comp_h0, weight_out, w_comp_out):
    src = edge_index[0]
    dst = edge_index[1]
    packed = (
        jnp.left_shift(edge_type, 28)
        | jnp.left_shift(src, 14)
        | dst
    )
    pad = E_PAD - N_EDGES
    packed2 = jnp.concatenate(
        [packed, jnp.full((pad,), PAD_DST, jnp.int32)]).reshape(-1, BLK)

    h1 = _layer(x, weight_in, w_comp_in, False, packed2)
    h2 = _layer(h1, weight_h0, w_comp_h0, True, packed2)

    w3 = jnp.broadcast_to(weight_out, (NUM_BASES, weight_out.shape[1], 32))
    t3 = _transform(h2, w3, w_comp_out, True)
    table3 = t3.reshape(NUM_RELS * N_NODES * 2, 16)
    agg3 = _aggregate(packed2, table3, 16, 2, 0)
    out = _relu(agg3[0])
    return out[:N_NODES, 0:1]


# speedup vs baseline: 1.0421x; 1.0272x over previous
import functools

import jax
import jax.numpy as jnp
from jax import lax
from jax.experimental import pallas as pl
from jax.experimental.pallas import tpu as pltpu
from jax.experimental.pallas import tpu_sc as plsc

N_NODES = 10000
N_EDGES = 320000
NUM_RELS = 4
NUM_BASES = 2

BLK = 64
N_TILES = 16
E_PAD = 327680
PAD_DST = 10008
N_ACC = 10112
ROWS_PER_TILE = N_ACC // N_TILES
MASK14 = (1 << 14) - 1


def _transform_body(relu_in, x_ref, bases_ref, wcomp_ref, out_ref):
    bf = jnp.bfloat16
    f32 = jnp.float32
    h = x_ref[...]
    if relu_in:
        h = jnp.maximum(h, 0.0)
    wc = wcomp_ref[0, 0].astype(bf).astype(f32)
    w = (wc[0] * bases_ref[0].astype(bf).astype(f32)
         + wc[1] * bases_ref[1].astype(bf).astype(f32))
    out_ref[0] = jnp.dot(h.astype(bf), w.astype(bf),
                         preferred_element_type=f32)


def _transform(h, bases, wcomp, relu_in):
    n, k = h.shape
    d_out = bases.shape[-1]
    nblk = 1000
    grid = (NUM_RELS, n // nblk)
    return pl.pallas_call(
        functools.partial(_transform_body, relu_in),
        grid=grid,
        in_specs=[
            pl.BlockSpec((nblk, k), lambda r, nb: (nb, 0)),
            pl.BlockSpec((NUM_BASES, k, d_out), lambda r, nb: (0, 0, 0)),
            pl.BlockSpec((1, 1, NUM_BASES), lambda r, nb: (r, 0, 0)),
        ],
        out_specs=pl.BlockSpec((1, nblk, d_out), lambda r, nb: (r, nb, 0)),
        out_shape=jax.ShapeDtypeStruct((NUM_RELS, n, d_out), jnp.float32),
    )(h, bases, wcomp.reshape(NUM_RELS, 1, NUM_BASES))


def _sum_relu_body(x_ref, o_ref):
    o_ref[...] = jnp.maximum(x_ref[0] + x_ref[1], 0.0)


def _sum_relu(x):
    return pl.pallas_call(
        _sum_relu_body,
        out_shape=jax.ShapeDtypeStruct(x.shape[1:], x.dtype),
    )(x)


NBUF = 8
AHEAD = 4


def _zero_acc(rows, acc, dh, base):
    nchunk = dh // 16

    def zero_rows(i, _):
        rows[0, i // nchunk, pl.ds((i % nchunk) * 16, 16)] = jnp.zeros(
            (16,), jnp.float32)
        return 0

    lax.fori_loop(0, BLK * nchunk, zero_rows, 0)
    nrep = ROWS_PER_TILE // BLK
    rem = ROWS_PER_TILE % BLK
    for rep in range(nrep):
        pltpu.sync_copy(rows.at[0], acc.at[pl.ds(base + rep * BLK, BLK)])
    if rem:
        pltpu.sync_copy(rows.at[0, pl.ds(0, rem)],
                        acc.at[pl.ds(base + nrep * BLK, rem)])


def _run_pass(table, idxbuf, dstbuf, rows, acc, gsem, ssem):
    nblocks = idxbuf.shape[0]

    def gather(j, b):
        pltpu.async_copy(table.at[idxbuf.at[j]], rows.at[b], gsem.at[b])

    def drain_gather(b):
        pltpu.make_async_copy(
            table.at[idxbuf.at[0]], rows.at[b], gsem.at[b]).wait()

    def scatter(j, b):
        pltpu.async_copy(rows.at[b], acc.at[dstbuf.at[j]], ssem.at[b],
                         add=True)

    def drain_scatter(b):
        pltpu.make_async_copy(
            rows.at[b], acc.at[dstbuf.at[0]], ssem.at[b]).wait()

    for b in range(AHEAD):
        gather(b, b)

    def outer(o, _):
        for k in range(NBUF):
            j = o * NBUF + k
            drain_gather(k)
            scatter(j, k)
            jn = j + AHEAD
            nxt = (k + AHEAD) % NBUF

            @pl.when(jn >= NBUF)
            def _():
                drain_scatter(nxt)

            @pl.when(jn < nblocks)
            def _():
                gather(jn, nxt)
        return 0

    lax.fori_loop(0, nblocks // NBUF, outer, 0)
    for i in range(AHEAD):
        drain_scatter((nblocks - AHEAD + i) % NBUF)


def _agg_body(dh, nsplit, edge_split, packed2, table, out, idxbuf, dstbuf,
              rows, acc, gsem, ssem):
    c = lax.axis_index("c")
    s = lax.axis_index("s")
    nblocks = idxbuf.shape[0]
    base = s * ROWS_PER_TILE
    nvec = BLK // 16

    _zero_acc(rows, acc, dh, base)
    plsc.subcore_barrier()

    roff = (c * N_TILES + s) * nblocks if edge_split else s * nblocks
    pltpu.sync_copy(packed2.at[pl.ds(roff, nblocks)], idxbuf)

    def calc_idx(i, _):
        j = i // nvec
        sl = pl.ds((i % nvec) * 16, 16)
        p = idxbuf[j, sl]
        dstbuf[j, sl] = p & MASK14
        row = ((p >> 28) * N_NODES + ((p >> 14) & MASK14)) * nsplit
        if not edge_split:
            row = row + c
        idxbuf[j, sl] = row
        return 0

    lax.fori_loop(0, nblocks * nvec, calc_idx, 0)

    _run_pass(table, idxbuf, dstbuf, rows, acc, gsem, ssem)
    plsc.subcore_barrier()
    pltpu.sync_copy(acc.at[pl.ds(base, ROWS_PER_TILE)],
                    out.at[c, pl.ds(base, ROWS_PER_TILE)])

    if nsplit == 4:
        def bump_idx(i, _):
            j = i // nvec
            sl = pl.ds((i % nvec) * 16, 16)
            idxbuf[j, sl] = idxbuf[j, sl] + 2
            return 0

        lax.fori_loop(0, nblocks * nvec, bump_idx, 0)
        _zero_acc(rows, acc, dh, base)
        plsc.subcore_barrier()
        _run_pass(table, idxbuf, dstbuf, rows, acc, gsem, ssem)
        plsc.subcore_barrier()
        pltpu.sync_copy(acc.at[pl.ds(base, ROWS_PER_TILE)],
                        out.at[2 + c, pl.ds(base, ROWS_PER_TILE)])


def _aggregate(packed2, table, dh, nsplit, edge_split=False):
    nout = 4 if nsplit == 4 else 2
    nblocks = (E_PAD // BLK) // (N_TILES * (2 if edge_split else 1))
    mesh = plsc.VectorSubcoreMesh(core_axis_name="c", subcore_axis_name="s")
    return pl.kernel(
        functools.partial(_agg_body, dh, nsplit, edge_split),
        out_type=jax.ShapeDtypeStruct((nout, N_ACC, dh), jnp.float32),
        mesh=mesh,
        scratch_types=[
            pltpu.VMEM((nblocks, BLK), jnp.int32),
            pltpu.VMEM((nblocks, BLK), jnp.int32),
            pltpu.VMEM((NBUF, BLK, dh), jnp.float32),
            pltpu.VMEM_SHARED((N_ACC, dh), jnp.float32),
            pltpu.SemaphoreType.DMA((NBUF,)),
            pltpu.SemaphoreType.DMA((NBUF,)),
        ],
        compiler_params=pltpu.CompilerParams(use_tc_tiling_on_sc=False),
    )(packed2, table)


def _layer(h, bases, wcomp, relu_in, packed2):
    d_out = bases.shape[-1]
    t = _transform(h, bases, wcomp, relu_in)
    table = t.reshape(NUM_RELS * N_NODES * 4, d_out // 4)
    agg = _aggregate(packed2, table, d_out // 4, 4)
    return jnp.concatenate([agg[q, :N_NODES] for q in range(4)], axis=1)


def kernel(x, edge_index, edge_type, weight_in, w_comp_in, weight_h0,
           w_comp_h0, weight_out, w_comp_out):
    src = edge_index[0]
    dst = edge_index[1]
    packed = (
        jnp.left_shift(edge_type, 28)
        | jnp.left_shift(src, 14)
        | dst
    )
    pad = E_PAD - N_EDGES
    packed2 = jnp.concatenate(
        [packed, jnp.full((pad,), PAD_DST, jnp.int32)]).reshape(-1, BLK)

    h1 = _layer(x, weight_in, w_comp_in, False, packed2)
    h2 = _layer(h1, weight_h0, w_comp_h0, True, packed2)

    w3 = jnp.broadcast_to(weight_out, (NUM_BASES, weight_out.shape[1], 16))
    t3 = _transform(h2, w3, w_comp_out, True)
    table3 = t3.reshape(NUM_RELS * N_NODES, 16)
    agg3 = _aggregate(packed2, table3, 16, 1, edge_split=True)
    out = _sum_relu(agg3)
    return out[:N_NODES, 0:1]
